# Initial kernel scaffold; baseline (speedup 1.0000x reference)
import jax
import jax.numpy as jnp
from jax.experimental import pallas as pl


def kernel(feat1, edge_index1, feat2, edge_index2, feat, edge_index, W1, b1, W2, b2):
    raise NotImplementedError("write your pallas kernel here")


# speedup vs baseline: 1.9385x; 1.9385x over previous
import functools

import jax
import jax.numpy as jnp
from jax import lax
from jax.experimental import pallas as pl
from jax.experimental.pallas import tpu as pltpu
from jax.experimental.pallas import tpu_sc as plsc

N = 10000
E = 320000
D = 128

NC = 2
NS = 16
CHUNK = 80
E_PER_SC = E // NC
E_PER_TILE = E_PER_SC // NS
NCHUNK = E_PER_TILE // CHUNK
NPAD = 10240
ROWS_PER_TILE = NPAD // NS
ZROWS = 128
DEGW = 16
OUTR = ROWS_PER_TILE * DEGW // 128

_MESH = dict(core_axis_name="c", subcore_axis_name="s",
             num_cores=NC, num_subcores=NS)


def _sc_propagate(ys, srcs, dsts, zrows_hbm):
    @functools.partial(
        pl.kernel,
        out_type=jax.ShapeDtypeStruct((3, NC, NPAD, D), jnp.float32),
        mesh=plsc.VectorSubcoreMesh(**_MESH),
        scratch_types=[
            pltpu.VMEM((CHUNK,), jnp.int32),
            pltpu.VMEM((CHUNK,), jnp.int32),
            pltpu.VMEM((CHUNK, D), jnp.float32),
            pltpu.VMEM((ZROWS, D), jnp.float32),
            pltpu.VMEM_SHARED((NPAD, D), jnp.float32),
            pltpu.SemaphoreType.DMA,
        ],
    )
    def prop_kernel(y0_hbm, y1_hbm, y2_hbm, s0, s1, s2, d0, d1, d2,
                    z_hbm, out_hbm, sidx, didx, rows, zrows, shared, sem):
        src_refs = (s0, s1, s2)
        dst_refs = (d0, d1, d2)
        c = lax.axis_index("c")
        s = lax.axis_index("s")
        rbase = s * ROWS_PER_TILE
        ebase = c * E_PER_SC + s * E_PER_TILE
        pltpu.sync_copy(z_hbm, zrows)
        for e, y_hbm in enumerate((y0_hbm, y1_hbm, y2_hbm)):
            for j in range(ROWS_PER_TILE // ZROWS):
                pltpu.sync_copy(zrows, shared.at[pl.ds(rbase + j * ZROWS, ZROWS)])
            plsc.subcore_barrier()

            src_hbm = src_refs[e]
            dst_hbm = dst_refs[e]

            def body(ch, carry):
                eb = ebase + ch * CHUNK
                pltpu.sync_copy(src_hbm.at[pl.ds(eb, CHUNK)], sidx)
                pltpu.sync_copy(dst_hbm.at[pl.ds(eb, CHUNK)], didx)
                pltpu.async_copy(y_hbm.at[sidx], rows, sem).wait()
                pltpu.sync_copy(rows, shared.at[didx], add=True)
                return carry

            lax.fori_loop(0, NCHUNK, body, None)
            plsc.subcore_barrier()
            for j in range(ROWS_PER_TILE // ZROWS):
                r0 = rbase + j * ZROWS
                pltpu.sync_copy(shared.at[pl.ds(r0, ZROWS)],
                                out_hbm.at[e, c, pl.ds(r0, ZROWS)])
            plsc.subcore_barrier()

    return prop_kernel(*ys, *srcs, *dsts, zrows_hbm)


_R = 400
_G = N // _R


def _norm(deg_ref, t):
    deg = deg_ref[0, 0, :, 0] + deg_ref[0, 1, :, 0]
    return jnp.where(deg > 0, lax.rsqrt(deg), 0.0)


def _tc_pre(feats, W1, dsrc):
    def body(f_ref, w_ref, d_ref, o_ref):
        nout = _norm(d_ref, 0)
        y = lax.dot(f_ref[0], w_ref[...], precision=lax.Precision.HIGHEST)
        o_ref[0] = y * nout[:, None]

    return pl.pallas_call(
        body,
        grid=(3, _G),
        in_specs=[
            pl.BlockSpec((1, _R, D), lambda e, i: (e, i, 0)),
            pl.BlockSpec((D, D), lambda e, i: (0, 0)),
            pl.BlockSpec((1, NC, _R, D), lambda e, i: (e, 0, i, 0)),
        ],
        out_specs=pl.BlockSpec((1, _R, D), lambda e, i: (e, i, 0)),
        out_shape=jax.ShapeDtypeStruct((3, N, D), jnp.float32),
    )(feats, W1, dsrc)


def _tc_mid(agg, ddst, dsrc, b1, W2):
    def body(a_ref, di_ref, ds_ref, b_ref, w_ref, o_ref):
        nin = _norm(di_ref, 0)
        nout = _norm(ds_ref, 0)
        a = a_ref[0, 0] + a_ref[0, 1]
        h = jnp.maximum(a * nin[:, None] + b_ref[...][None, :], 0.0)
        y = lax.dot(h, w_ref[...], precision=lax.Precision.HIGHEST)
        o_ref[0] = y * nout[:, None]

    return pl.pallas_call(
        body,
        grid=(3, _G),
        in_specs=[
            pl.BlockSpec((1, NC, _R, D), lambda e, i: (e, 0, i, 0)),
            pl.BlockSpec((1, NC, _R, D), lambda e, i: (e, 0, i, 0)),
            pl.BlockSpec((1, NC, _R, D), lambda e, i: (e, 0, i, 0)),
            pl.BlockSpec((D,), lambda e, i: (0,)),
            pl.BlockSpec((D, D), lambda e, i: (0, 0)),
        ],
        out_specs=pl.BlockSpec((1, _R, D), lambda e, i: (e, i, 0)),
        out_shape=jax.ShapeDtypeStruct((3, N, D), jnp.float32),
    )(agg, ddst, dsrc, b1, W2)


def _tc_fin(agg, ddst, b2):
    def body(a_ref, di_ref, b_ref, o_ref):
        nin = _norm(di_ref, 0)
        a = a_ref[0, 0] + a_ref[0, 1]
        o_ref[0] = a * nin[:, None] + b_ref[...][None, :]

    return pl.pallas_call(
        body,
        grid=(3, _G),
        in_specs=[
            pl.BlockSpec((1, NC, _R, D), lambda e, i: (e, 0, i, 0)),
            pl.BlockSpec((1, NC, _R, D), lambda e, i: (e, 0, i, 0)),
            pl.BlockSpec((D,), lambda e, i: (0,)),
        ],
        out_specs=pl.BlockSpec((1, _R, D), lambda e, i: (e, i, 0)),
        out_shape=jax.ShapeDtypeStruct((3, N, D), jnp.float32),
    )(agg, ddst, b2)


def kernel(feat1, edge_index1, feat2, edge_index2, feat, edge_index,
           W1, b1, W2, b2):
    srcs = (edge_index1[0], edge_index2[0], edge_index[0])
    dsts = (edge_index1[1], edge_index2[1], edge_index[1])
    feats = jnp.stack([feat1, feat2, feat])
    zrows_hbm = jnp.zeros((ZROWS, D), jnp.float32)
    ones_feat = jnp.ones((N, D), jnp.float32)

    ones3 = (ones_feat, ones_feat, ones_feat)
    dsrc = _sc_propagate(ones3, srcs, srcs, zrows_hbm)
    ddst = _sc_propagate(ones3, dsts, dsts, zrows_hbm)

    y = _tc_pre(feats, W1, dsrc)
    agg = _sc_propagate((y[0], y[1], y[2]), srcs, dsts, zrows_hbm)
    y2 = _tc_mid(agg, ddst, dsrc, b1, W2)
    agg2 = _sc_propagate((y2[0], y2[1], y2[2]), srcs, dsts, zrows_hbm)
    z = _tc_fin(agg2, ddst, b2)
    return z[0], z[1], z[2]


# speedup vs baseline: 3.9975x; 2.0622x over previous
import functools

import jax
import jax.numpy as jnp
from jax import lax
from jax.experimental import pallas as pl
from jax.experimental.pallas import tpu as pltpu
from jax.experimental.pallas import tpu_sc as plsc

N = 10000
E = 320000
D = 128

NC = 2
NS = 16
CHUNK = 40
E_PER_SC = E // NC
E_PER_TILE = E_PER_SC // NS
NCHUNK = E_PER_TILE // CHUNK
NPAD = 10240
ROWS_PER_TILE = NPAD // NS
ZROWS = 64
NB = 5
DEGW = 16
OUTR = ROWS_PER_TILE * DEGW // 128

_MESH = dict(core_axis_name="c", subcore_axis_name="s",
             num_cores=NC, num_subcores=NS)


def _sc_propagate(ys, srcs, dsts, zrows_hbm, mode_hbm):
    @functools.partial(
        pl.kernel,
        out_type=jax.ShapeDtypeStruct((3, NC, NPAD, D), jnp.float32),
        mesh=plsc.VectorSubcoreMesh(**_MESH),
        scratch_types=[
            pltpu.VMEM((NB, CHUNK), jnp.int32),
            pltpu.VMEM((NB, CHUNK), jnp.int32),
            pltpu.VMEM((NB, CHUNK, D), jnp.float32),
            pltpu.VMEM((ZROWS, D), jnp.float32),
            pltpu.VMEM((16,), jnp.int32),
            pltpu.VMEM_SHARED((NPAD, D), jnp.float32),
            pltpu.SemaphoreType.DMA,
            pltpu.SemaphoreType.DMA,
            pltpu.SemaphoreType.DMA,
        ],
    )
    def prop_kernel(y0_hbm, y1_hbm, y2_hbm, s0, s1, s2, d0, d1, d2,
                    z_hbm, mode_ref, out_hbm,
                    sidx, didx, rows, zrows, mode_v, shared,
                    isem, gsem, ssem):
        src_refs = (s0, s1, s2)
        dst_refs = (d0, d1, d2)
        c = lax.axis_index("c")
        s = lax.axis_index("s")
        rbase = s * ROWS_PER_TILE
        ebase = c * E_PER_SC + s * E_PER_TILE
        pltpu.sync_copy(z_hbm, zrows)
        pltpu.sync_copy(mode_ref, mode_v)
        deg_mode = mode_v[...][0] == 1

        onerow = jnp.ones((16,), jnp.float32)

        @pl.when(deg_mode)
        def _fill_ones():
            def fill(i, carry):
                for j in range(D // 16):
                    rows[i // CHUNK, i % CHUNK, pl.ds(j * 16, 16)] = onerow
                return carry
            lax.fori_loop(0, NB * CHUNK, fill, None)

        for e, y_hbm in enumerate((y0_hbm, y1_hbm, y2_hbm)):
            for j in range(ROWS_PER_TILE // ZROWS):
                pltpu.sync_copy(zrows, shared.at[pl.ds(rbase + j * ZROWS, ZROWS)])
            plsc.subcore_barrier()

            src_hbm = src_refs[e]
            dst_hbm = dst_refs[e]

            def batch(bi, carry):
                base = ebase + bi * (NB * CHUNK)
                idescs = []
                for b in range(NB):
                    idescs.append(pltpu.async_copy(
                        src_hbm.at[pl.ds(base + b * CHUNK, CHUNK)],
                        sidx.at[b], isem))
                    idescs.append(pltpu.async_copy(
                        dst_hbm.at[pl.ds(base + b * CHUNK, CHUNK)],
                        didx.at[b], isem))
                for desc in idescs:
                    desc.wait()

                @pl.when(jnp.logical_not(deg_mode))
                def _gather():
                    gdescs = [pltpu.async_copy(y_hbm.at[sidx.at[b]],
                                               rows.at[b], gsem)
                              for b in range(NB)]
                    for desc in gdescs:
                        desc.wait()

                sdescs = [pltpu.async_copy(rows.at[b],
                                           shared.at[didx.at[b]], ssem,
                                           add=True)
                          for b in range(NB)]
                for desc in sdescs:
                    desc.wait()
                return carry

            lax.fori_loop(0, NCHUNK // NB, batch, None)
            plsc.subcore_barrier()
            for j in range(ROWS_PER_TILE // ZROWS):
                r0 = rbase + j * ZROWS
                pltpu.sync_copy(shared.at[pl.ds(r0, ZROWS)],
                                out_hbm.at[e, c, pl.ds(r0, ZROWS)])
            plsc.subcore_barrier()

    return prop_kernel(*ys, *srcs, *dsts, zrows_hbm, mode_hbm)


_R = 400
_G = N // _R


def _norm(deg_ref, t):
    deg = deg_ref[0, 0, :, 0] + deg_ref[0, 1, :, 0]
    return jnp.where(deg > 0, lax.rsqrt(deg), 0.0)


def _tc_pre(feats, W1, dsrc):
    def body(f_ref, w_ref, d_ref, o_ref):
        nout = _norm(d_ref, 0)
        y = lax.dot(f_ref[0], w_ref[...], precision=lax.Precision.HIGHEST)
        o_ref[0] = y * nout[:, None]

    return pl.pallas_call(
        body,
        grid=(3, _G),
        in_specs=[
            pl.BlockSpec((1, _R, D), lambda e, i: (e, i, 0)),
            pl.BlockSpec((D, D), lambda e, i: (0, 0)),
            pl.BlockSpec((1, NC, _R, D), lambda e, i: (e, 0, i, 0)),
        ],
        out_specs=pl.BlockSpec((1, _R, D), lambda e, i: (e, i, 0)),
        out_shape=jax.ShapeDtypeStruct((3, N, D), jnp.float32),
    )(feats, W1, dsrc)


def _tc_mid(agg, ddst, dsrc, b1, W2):
    def body(a_ref, di_ref, ds_ref, b_ref, w_ref, o_ref):
        nin = _norm(di_ref, 0)
        nout = _norm(ds_ref, 0)
        a = a_ref[0, 0] + a_ref[0, 1]
        h = jnp.maximum(a * nin[:, None] + b_ref[...][None, :], 0.0)
        y = lax.dot(h, w_ref[...], precision=lax.Precision.HIGHEST)
        o_ref[0] = y * nout[:, None]

    return pl.pallas_call(
        body,
        grid=(3, _G),
        in_specs=[
            pl.BlockSpec((1, NC, _R, D), lambda e, i: (e, 0, i, 0)),
            pl.BlockSpec((1, NC, _R, D), lambda e, i: (e, 0, i, 0)),
            pl.BlockSpec((1, NC, _R, D), lambda e, i: (e, 0, i, 0)),
            pl.BlockSpec((D,), lambda e, i: (0,)),
            pl.BlockSpec((D, D), lambda e, i: (0, 0)),
        ],
        out_specs=pl.BlockSpec((1, _R, D), lambda e, i: (e, i, 0)),
        out_shape=jax.ShapeDtypeStruct((3, N, D), jnp.float32),
    )(agg, ddst, dsrc, b1, W2)


def _tc_fin(agg, ddst, b2):
    def body(a_ref, di_ref, b_ref, o_ref):
        nin = _norm(di_ref, 0)
        a = a_ref[0, 0] + a_ref[0, 1]
        o_ref[0] = a * nin[:, None] + b_ref[...][None, :]

    return pl.pallas_call(
        body,
        grid=(3, _G),
        in_specs=[
            pl.BlockSpec((1, NC, _R, D), lambda e, i: (e, 0, i, 0)),
            pl.BlockSpec((1, NC, _R, D), lambda e, i: (e, 0, i, 0)),
            pl.BlockSpec((D,), lambda e, i: (0,)),
        ],
        out_specs=pl.BlockSpec((1, _R, D), lambda e, i: (e, i, 0)),
        out_shape=jax.ShapeDtypeStruct((3, N, D), jnp.float32),
    )(agg, ddst, b2)


def kernel(feat1, edge_index1, feat2, edge_index2, feat, edge_index,
           W1, b1, W2, b2):
    srcs = (edge_index1[0], edge_index2[0], edge_index[0])
    dsts = (edge_index1[1], edge_index2[1], edge_index[1])
    feats = jnp.stack([feat1, feat2, feat])
    zrows_hbm = jnp.zeros((ZROWS, D), jnp.float32)
    ones_feat = jnp.ones((N, D), jnp.float32)

    mode_deg = jnp.ones((16,), jnp.int32)
    mode_agg = jnp.zeros((16,), jnp.int32)
    ones3 = (ones_feat, ones_feat, ones_feat)
    dsrc = _sc_propagate(ones3, srcs, srcs, zrows_hbm, mode_deg)
    mode_deg2 = mode_deg + (dsrc[0, 0, 0, :16] * 0.0).astype(jnp.int32)
    ddst = _sc_propagate(ones3, dsts, dsts, zrows_hbm, mode_deg2)

    y = _tc_pre(feats, W1, dsrc)
    agg = _sc_propagate((y[0], y[1], y[2]), srcs, dsts, zrows_hbm, mode_agg)
    y2 = _tc_mid(agg, ddst, dsrc, b1, W2)
    agg2 = _sc_propagate((y2[0], y2[1], y2[2]), srcs, dsts, zrows_hbm, mode_agg)
    z = _tc_fin(agg2, ddst, b2)
    return z[0], z[1], z[2]


# speedup vs baseline: 4.5059x; 1.1272x over previous
import functools

import jax
import jax.numpy as jnp
from jax import lax
from jax.experimental import pallas as pl
from jax.experimental.pallas import tpu as pltpu
from jax.experimental.pallas import tpu_sc as plsc

N = 10000
E = 320000
D = 128

NC = 2
NS = 16
CHUNK = 40
E_PER_SC = E // NC
E_PER_TILE = E_PER_SC // NS
NCHUNK = E_PER_TILE // CHUNK
NPAD = 10240
ROWS_PER_TILE = NPAD // NS
ZROWS = 64
NB = 2
DEGW = 16
OUTR = ROWS_PER_TILE * DEGW // 128

_MESH = dict(core_axis_name="c", subcore_axis_name="s",
             num_cores=NC, num_subcores=NS)


def _sc_propagate(ys, srcs, dsts, zrows_hbm, mode_hbm):
    @functools.partial(
        pl.kernel,
        out_type=jax.ShapeDtypeStruct((3, NC, NPAD, D), jnp.float32),
        mesh=plsc.VectorSubcoreMesh(**_MESH),
        scratch_types=[
            pltpu.VMEM((4, NB, CHUNK), jnp.int32),
            pltpu.VMEM((4, NB, CHUNK), jnp.int32),
            pltpu.VMEM((2, NB, CHUNK, D), jnp.float32),
            pltpu.VMEM((ZROWS, D), jnp.float32),
            pltpu.VMEM((16,), jnp.int32),
            pltpu.VMEM_SHARED((NPAD, D), jnp.float32),
        ] + [pltpu.SemaphoreType.DMA] * 8,
    )
    def prop_kernel(y0_hbm, y1_hbm, y2_hbm, s0, s1, s2, d0, d1, d2,
                    z_hbm, mode_ref, out_hbm,
                    sidx, didx, rows, zrows, mode_v, shared,
                    i0sem, i1sem, i2sem, i3sem, g0sem, g1sem, s0sem, s1sem):
        src_refs = (s0, s1, s2)
        dst_refs = (d0, d1, d2)
        isems = (i0sem, i1sem, i2sem, i3sem)
        gsems = (g0sem, g1sem)
        ssems = (s0sem, s1sem)
        c = lax.axis_index("c")
        s = lax.axis_index("s")
        rbase = s * ROWS_PER_TILE
        ebase = c * E_PER_SC + s * E_PER_TILE
        pltpu.sync_copy(z_hbm, zrows)
        pltpu.sync_copy(mode_ref, mode_v)
        deg_mode = mode_v[...][0] == 1
        gather_mode = jnp.logical_not(deg_mode)

        onerow = jnp.ones((16,), jnp.float32)

        @pl.when(deg_mode)
        def _fill_ones():
            def fill(i, carry):
                for j in range(D // 16):
                    rows[i // (NB * CHUNK), (i // CHUNK) % NB, i % CHUNK,
                         pl.ds(j * 16, 16)] = onerow
                return carry
            lax.fori_loop(0, 2 * NB * CHUNK, fill, None)

        BB = NB * CHUNK
        NBATCH = E_PER_TILE // BB

        for e, y_hbm in enumerate((y0_hbm, y1_hbm, y2_hbm)):
            for j in range(ROWS_PER_TILE // ZROWS):
                pltpu.sync_copy(zrows, shared.at[pl.ds(rbase + j * ZROWS, ZROWS)])
            plsc.subcore_barrier()

            src_hbm = src_refs[e]
            dst_hbm = dst_refs[e]

            def fire_idx(bi, r):
                for b in range(NB):
                    off = ebase + bi * BB + b * CHUNK
                    pltpu.async_copy(src_hbm.at[pl.ds(off, CHUNK)],
                                     sidx.at[r, b], isems[r])
                    pltpu.async_copy(dst_hbm.at[pl.ds(off, CHUNK)],
                                     didx.at[r, b], isems[r])

            def drain_idx(r):
                for b in range(NB):
                    pltpu.make_async_copy(src_hbm.at[pl.ds(ebase, CHUNK)],
                                          sidx.at[r, b], isems[r]).wait()
                    pltpu.make_async_copy(dst_hbm.at[pl.ds(ebase, CHUNK)],
                                          didx.at[r, b], isems[r]).wait()

            def fire_gather(r, p):
                @pl.when(gather_mode)
                def _():
                    for b in range(NB):
                        pltpu.async_copy(y_hbm.at[sidx.at[r, b]],
                                         rows.at[p, b], gsems[p])

            def drain_gather(r, p):
                @pl.when(gather_mode)
                def _():
                    for b in range(NB):
                        pltpu.make_async_copy(y_hbm.at[sidx.at[r, b]],
                                              rows.at[p, b], gsems[p]).wait()

            def fire_scatter(r, p):
                for b in range(NB):
                    pltpu.async_copy(rows.at[p, b], shared.at[didx.at[r, b]],
                                     ssems[p], add=True)

            def drain_scatter(r, p):
                for b in range(NB):
                    pltpu.make_async_copy(rows.at[p, b],
                                          shared.at[didx.at[r, b]],
                                          ssems[p]).wait()

            def step(i, bi, first, last):
                r, p = i % 4, i % 2
                r1, p1 = (i + 1) % 4, (i + 1) % 2
                r2 = (i + 2) % 4
                drain_gather(r, p)
                fire_scatter(r, p)
                drain_idx(r1)
                if not first:
                    drain_scatter((i - 1) % 4, p1)
                if not last:
                    fire_idx(bi + 2, r2)
                fire_gather(r1, p1)

            fire_idx(0, 0)
            fire_idx(1, 1)
            drain_idx(0)
            fire_gather(0, 0)
            step(0, 0, True, False)
            step(1, 1, False, False)
            step(2, 2, False, False)
            step(3, 3, False, False)

            def body(i4, carry):
                bi = i4 * 4
                step(0, bi, False, False)
                step(1, bi + 1, False, False)
                step(2, bi + 2, False, False)
                step(3, bi + 3, False, False)
                return carry

            lax.fori_loop(1, (NBATCH - 5) // 4, body, None)
            step(0, 120, False, False)
            step(1, 121, False, False)
            step(2, 122, False, False)
            step(3, 123, False, True)
            drain_gather(0, 0)
            fire_scatter(0, 0)
            drain_scatter(3 % 4, 1)
            drain_scatter(0, 0)
            plsc.subcore_barrier()
            for j in range(ROWS_PER_TILE // ZROWS):
                r0 = rbase + j * ZROWS
                pltpu.sync_copy(shared.at[pl.ds(r0, ZROWS)],
                                out_hbm.at[e, c, pl.ds(r0, ZROWS)])
            plsc.subcore_barrier()

    return prop_kernel(*ys, *srcs, *dsts, zrows_hbm, mode_hbm)


_R = 400
_G = N // _R


def _norm(deg_ref, t):
    deg = deg_ref[0, 0, :, 0] + deg_ref[0, 1, :, 0]
    return jnp.where(deg > 0, lax.rsqrt(deg), 0.0)


def _tc_pre(feats, W1, dsrc):
    def body(f_ref, w_ref, d_ref, o_ref):
        nout = _norm(d_ref, 0)
        y = lax.dot(f_ref[0], w_ref[...], precision=lax.Precision.HIGHEST)
        o_ref[0] = y * nout[:, None]

    return pl.pallas_call(
        body,
        grid=(3, _G),
        in_specs=[
            pl.BlockSpec((1, _R, D), lambda e, i: (e, i, 0)),
            pl.BlockSpec((D, D), lambda e, i: (0, 0)),
            pl.BlockSpec((1, NC, _R, D), lambda e, i: (e, 0, i, 0)),
        ],
        out_specs=pl.BlockSpec((1, _R, D), lambda e, i: (e, i, 0)),
        out_shape=jax.ShapeDtypeStruct((3, N, D), jnp.float32),
    )(feats, W1, dsrc)


def _tc_mid(agg, ddst, dsrc, b1, W2):
    def body(a_ref, di_ref, ds_ref, b_ref, w_ref, o_ref):
        nin = _norm(di_ref, 0)
        nout = _norm(ds_ref, 0)
        a = a_ref[0, 0] + a_ref[0, 1]
        h = jnp.maximum(a * nin[:, None] + b_ref[...][None, :], 0.0)
        y = lax.dot(h, w_ref[...], precision=lax.Precision.HIGHEST)
        o_ref[0] = y * nout[:, None]

    return pl.pallas_call(
        body,
        grid=(3, _G),
        in_specs=[
            pl.BlockSpec((1, NC, _R, D), lambda e, i: (e, 0, i, 0)),
            pl.BlockSpec((1, NC, _R, D), lambda e, i: (e, 0, i, 0)),
            pl.BlockSpec((1, NC, _R, D), lambda e, i: (e, 0, i, 0)),
            pl.BlockSpec((D,), lambda e, i: (0,)),
            pl.BlockSpec((D, D), lambda e, i: (0, 0)),
        ],
        out_specs=pl.BlockSpec((1, _R, D), lambda e, i: (e, i, 0)),
        out_shape=jax.ShapeDtypeStruct((3, N, D), jnp.float32),
    )(agg, ddst, dsrc, b1, W2)


def _tc_fin(agg, ddst, b2):
    def body(a_ref, di_ref, b_ref, o_ref):
        nin = _norm(di_ref, 0)
        a = a_ref[0, 0] + a_ref[0, 1]
        o_ref[0] = a * nin[:, None] + b_ref[...][None, :]

    return pl.pallas_call(
        body,
        grid=(3, _G),
        in_specs=[
            pl.BlockSpec((1, NC, _R, D), lambda e, i: (e, 0, i, 0)),
            pl.BlockSpec((1, NC, _R, D), lambda e, i: (e, 0, i, 0)),
            pl.BlockSpec((D,), lambda e, i: (0,)),
        ],
        out_specs=pl.BlockSpec((1, _R, D), lambda e, i: (e, i, 0)),
        out_shape=jax.ShapeDtypeStruct((3, N, D), jnp.float32),
    )(agg, ddst, b2)


def kernel(feat1, edge_index1, feat2, edge_index2, feat, edge_index,
           W1, b1, W2, b2):
    pad = jnp.zeros((128,), jnp.int32)
    srcs = tuple(jnp.concatenate([e[0], pad]) for e in
                 (edge_index1, edge_index2, edge_index))
    dsts = tuple(jnp.concatenate([e[1], pad]) for e in
                 (edge_index1, edge_index2, edge_index))
    feats = jnp.stack([feat1, feat2, feat])
    zrows_hbm = jnp.zeros((ZROWS, D), jnp.float32)
    ones_feat = jnp.ones((N, D), jnp.float32)

    mode_deg = jnp.ones((16,), jnp.int32)
    mode_agg = jnp.zeros((16,), jnp.int32)
    ones3 = (ones_feat, ones_feat, ones_feat)
    dsrc = _sc_propagate(ones3, srcs, srcs, zrows_hbm, mode_deg)
    mode_deg2 = mode_deg + (dsrc[0, 0, 0, :16] * 0.0).astype(jnp.int32)
    ddst = _sc_propagate(ones3, dsts, dsts, zrows_hbm, mode_deg2)

    y = _tc_pre(feats, W1, dsrc)
    agg = _sc_propagate((y[0], y[1], y[2]), srcs, dsts, zrows_hbm, mode_agg)
    y2 = _tc_mid(agg, ddst, dsrc, b1, W2)
    agg2 = _sc_propagate((y2[0], y2[1], y2[2]), srcs, dsts, zrows_hbm, mode_agg)
    z = _tc_fin(agg2, ddst, b2)
    return z[0], z[1], z[2]
